# baseline (speedup 1.0000x reference)
import functools

import jax
import jax.numpy as jnp
from jax import lax
from jax.experimental import pallas as pl
from jax.experimental.pallas import tpu as pltpu
from jax.experimental.pallas import tpu_sc as plsc

N_OBS = 1000000
NC = 2
NS = 16
NW = NC * NS
LANES = 16

CHUNK = 3968
NCHUNKS = (N_OBS + CHUNK - 1) // CHUNK
EXT_WORDS = 2000 * 8


def _sc_body(p2d, ptidx, imidx, dep, par, px, py, pz, ext,
             out,
             ptidx_v, imidx_v, pxv, pyv, pzv, p2dv, depv,
             out_v, ext_v, par_v, semi, semg):
    wid = lax.axis_index("s") * NC + lax.axis_index("c")
    pltpu.sync_copy(par, par_v)
    pltpu.sync_copy(ext, ext_v)
    fx = par_v[0]
    fy = par_v[1]
    k1 = par_v[2]
    k2 = par_v[3]
    ppx = par_v[4]
    ppy = par_v[5]

    iota = lax.iota(jnp.int32, LANES)
    c0 = iota * 0
    c1 = c0 + 1
    c2 = c0 + 2

    nj = (NCHUNKS - 1 - wid) // NW + 1

    def chunk_body(j, carry):
        c = wid + j * NW
        base = jnp.minimum(c * CHUNK, N_OBS - CHUNK)
        cpa = pltpu.async_copy(ptidx.at[pl.ds(base, CHUNK)], ptidx_v, semi)
        cpb = pltpu.async_copy(imidx.at[pl.ds(base, CHUNK)], imidx_v, semi)
        cpc = pltpu.async_copy(p2d.at[pl.ds(base * 2, CHUNK * 2)], p2dv, semi)
        cpd = pltpu.async_copy(dep.at[pl.ds(base, CHUNK)], depv, semi)
        cpa.wait()
        cp1 = pltpu.async_copy(px.at[ptidx_v], pxv, semg)
        cp2 = pltpu.async_copy(py.at[ptidx_v], pyv, semg)
        cp3 = pltpu.async_copy(pz.at[ptidx_v], pzv, semg)
        cpb.wait()
        cpc.wait()
        cpd.wait()
        cp1.wait()
        cp2.wait()
        cp3.wait()

        def grp(g, gc):
            b = g * LANES
            sl = pl.ds(b, LANES)
            rI = b + iota
            vx = pxv[sl]
            vy = pyv[sl]
            vz = pzv[sl]
            im8 = imidx_v[sl] * 8
            qw = plsc.load_gather(ext_v, [im8])
            qx = plsc.load_gather(ext_v, [im8 + 1])
            qy = plsc.load_gather(ext_v, [im8 + 2])
            qz = plsc.load_gather(ext_v, [im8 + 3])
            tx = plsc.load_gather(ext_v, [im8 + 4])
            ty = plsc.load_gather(ext_v, [im8 + 5])
            tz = plsc.load_gather(ext_v, [im8 + 6])
            rI2 = rI * 2
            ox = plsc.load_gather(p2dv, [rI2])
            oy = plsc.load_gather(p2dv, [rI2 + 1])
            dref = depv[sl]

            s = qw * qw + qx * qx + qy * qy + qz * qz
            uvx = qy * vz - qz * vy
            uvy = qz * vx - qx * vz
            uvz = qx * vy - qy * vx
            uux = qy * uvz - qz * uvy
            uuy = qz * uvx - qx * uvz
            uuz = qx * uvy - qy * uvx
            inv2 = 2.0 / s
            pcx = vx + inv2 * (qw * uvx + uux) + tx
            pcy = vy + inv2 * (qw * uvy + uuy) + ty
            pcz = vz + inv2 * (qw * uvz + uuz) + tz
            rcp = 1.0 / (pcz + 1e-6)
            xn = pcx * rcp
            yn = pcy * rcp
            r2 = xn * xn + yn * yn
            dist = 1.0 + r2 * (k1 + k2 * r2)
            rI3 = rI * 3
            plsc.store_scatter(out_v, [rI3], fx * xn * dist + ppx - ox)
            plsc.store_scatter(out_v, [rI3 + 1], fy * yn * dist + ppy - oy)
            plsc.store_scatter(out_v, [rI3 + 2], rcp - dref)
            return gc

        lax.fori_loop(0, CHUNK // LANES, grp, 0)
        pltpu.sync_copy(out_v, out.at[pl.ds(base * 3, CHUNK * 3)])
        return carry

    lax.fori_loop(0, nj, chunk_body, 0)


_sc_call = functools.partial(
    pl.kernel,
    out_type=jax.ShapeDtypeStruct((N_OBS * 3,), jnp.float32),
    mesh=plsc.VectorSubcoreMesh(core_axis_name="c", subcore_axis_name="s"),
    compiler_params=pltpu.CompilerParams(needs_layout_passes=False),
    scratch_types=[
        pltpu.VMEM((CHUNK,), jnp.int32),
        pltpu.VMEM((CHUNK,), jnp.int32),
        pltpu.VMEM((CHUNK,), jnp.float32),
        pltpu.VMEM((CHUNK,), jnp.float32),
        pltpu.VMEM((CHUNK,), jnp.float32),
        pltpu.VMEM((CHUNK * 2,), jnp.float32),
        pltpu.VMEM((CHUNK,), jnp.float32),
        pltpu.VMEM((CHUNK * 3,), jnp.float32),
        pltpu.VMEM((EXT_WORDS,), jnp.float32),
        pltpu.VMEM((6, LANES), jnp.float32),
        pltpu.SemaphoreType.DMA,
        pltpu.SemaphoreType.DMA,
    ],
)(_sc_body)


def kernel(points_2d, image_indices, camera_indices, point_indices,
           camera_pps, depths_ref, extrinsics, intrinsics, points_3d):
    ptidx = point_indices.astype(jnp.int32)
    imidx = image_indices.astype(jnp.int32)
    px = points_3d[:, 0]
    py = points_3d[:, 1]
    pz = points_3d[:, 2]
    ext = jnp.pad(extrinsics, ((0, 0), (0, 1))).reshape(-1)
    par = jnp.tile(
        jnp.concatenate([intrinsics[0], camera_pps[0]])[:, None], (1, LANES))
    out = _sc_call(points_2d.reshape(-1), ptidx, imidx, depths_ref, par,
                   px, py, pz, ext)
    return out.reshape(N_OBS, 3)
